# baseline (speedup 1.0000x reference)
import functools

import numpy as np
import jax
import jax.numpy as jnp
from jax import lax
from jax.experimental import pallas as pl
from jax.experimental.pallas import tpu as pltpu
from jax.experimental.pallas import tpu_sc as plsc

_LANES = 16
_CHUNK = 8
_NRING = 3
_CUNROLL = 2


def _positional_encoding_np(seq_len, d_model):
    pos = np.arange(seq_len, dtype=np.float32)[:, None]
    i = np.arange(d_model // 2, dtype=np.float32)[None, :]
    div = np.exp(-(2.0 * i / d_model) * np.log(10000.0))
    ang = pos * div
    pe = np.zeros((seq_len, d_model), dtype=np.float32)
    pe[:, 0::2] = np.sin(ang)
    pe[:, 1::2] = np.cos(ang)
    return pe


def _pe_packed_words(seq_len, d_model):
    flat = _positional_encoding_np(seq_len, d_model).reshape(-1, 32)
    bf = flat.astype(jnp.bfloat16).view(np.uint16).astype(np.uint32)
    words = bf[:, :16] | (bf[:, 16:] << 16)
    return words.astype(np.int32).reshape(-1)


@functools.lru_cache(maxsize=None)
def _build(batch, seq, vocab, d):
    info = plsc.get_sparse_core_info()
    nc, ns = info.num_cores, info.num_subcores
    nw = nc * ns
    pos_per_w = seq // nw
    n_grp = pos_per_w // _CHUNK
    n_pair = d // (2 * _LANES)
    scale = float(np.sqrt(d))
    mesh = plsc.VectorSubcoreMesh(core_axis_name="c", subcore_axis_name="s")

    scratch = [
        pltpu.VMEM((batch, pos_per_w), jnp.int32),
        pltpu.VMEM((_NRING * batch, _CHUNK, d), jnp.float32),
        pltpu.VMEM((_CHUNK * d // 2,), jnp.int32),
        pltpu.VMEM((_CHUNK * d // 2,), jnp.int32),
    ] + [pltpu.SemaphoreType.DMA for _ in range(_NRING * 2 + 2)]

    @functools.partial(
        pl.kernel,
        mesh=mesh,
        out_type=jax.ShapeDtypeStruct((batch, seq, d), jnp.float32),
        scratch_types=scratch,
    )
    def emb_kernel(x_hbm, table_hbm, pe_hbm, out_hbm, idx_v, rows_v,
                   pe_v0, pe_v1, *sems):
        pe_bufs = (pe_v0, pe_v1)
        g_sem = sems[0:_NRING]
        st_sem = sems[_NRING:2 * _NRING]
        pe_sem = sems[2 * _NRING:]

        wid = lax.axis_index("s") * nc + lax.axis_index("c")
        pos_base = wid * pos_per_w

        def start_gathers(g):
            p = g % _NRING
            return [
                pltpu.async_copy(
                    table_hbm.at[idx_v.at[b, pl.ds(g * _CHUNK, _CHUNK)]],
                    rows_v.at[p * batch + b], g_sem[p])
                for b in range(batch)
            ]

        def start_pe(g):
            off = (pos_base + g * _CHUNK) * (d // 2)
            return pltpu.async_copy(
                pe_hbm.at[pl.ds(off, _CHUNK * d // 2)],
                pe_bufs[g % 2], pe_sem[g % 2])

        for b in range(batch):
            pltpu.sync_copy(x_hbm.at[b, pl.ds(pos_base, pos_per_w)],
                            idx_v.at[b])
        g_h = {0: start_gathers(0), 1: start_gathers(1)}
        pe_h = {0: start_pe(0)}
        if n_grp > 1:
            pe_h[1] = start_pe(1)

        st_h = {}
        for g in range(n_grp):
            p = g % _NRING
            q = g % 2
            if g + 2 < n_grp:
                if g - 1 >= 0:
                    for h in st_h[g - 1]:
                        h.wait()
                g_h[g + 2] = start_gathers(g + 2)
            pe_h[g].wait()
            for h in g_h[g]:
                h.wait()

            def row_body(r, _):
                row_base = r * d

                @plsc.parallel_loop(0, n_pair, 1, unroll=_CUNROLL)
                def pair_body(c):
                    off = c * (2 * _LANES)
                    pw = pe_bufs[q][pl.ds(row_base // 2 + c * _LANES, _LANES)]
                    pva = lax.bitcast_convert_type(
                        lax.shift_left(pw, 16), jnp.float32)
                    pvb = lax.bitcast_convert_type(
                        lax.bitwise_and(pw, jnp.int32(-65536)), jnp.float32)
                    for half, pv in ((0, pva), (1, pvb)):
                        o = off + half * _LANES
                        for b in range(batch):
                            k = p * batch + b
                            rv = rows_v[k, r, pl.ds(o, _LANES)]
                            rows_v[k, r, pl.ds(o, _LANES)] = rv * scale + pv

                return 0

            lax.fori_loop(0, _CHUNK, row_body, 0)

            hs = []
            for b in range(batch):
                hs.append(pltpu.async_copy(
                    rows_v.at[p * batch + b],
                    out_hbm.at[b, pl.ds(pos_base + g * _CHUNK, _CHUNK)],
                    st_sem[p]))
            st_h[g] = hs
            if g + 2 < n_grp:
                pe_h[g + 2] = start_pe(g + 2)

        for g in range(max(0, n_grp - _NRING), n_grp):
            for h in st_h[g]:
                h.wait()

    return emb_kernel


def kernel(x, table):
    b, s = x.shape
    v, d = table.shape
    pe = jnp.asarray(_pe_packed_words(s, d))
    return _build(b, s, v, d)(x, table, pe)


# speedup vs baseline: 1.0196x; 1.0196x over previous
import functools

import numpy as np
import jax
import jax.numpy as jnp
from jax import lax
from jax.experimental import pallas as pl
from jax.experimental.pallas import tpu as pltpu
from jax.experimental.pallas import tpu_sc as plsc

_LANES = 16
_CHUNK = 8
_NRING = 3
_CUNROLL = 2


def _positional_encoding_np(seq_len, d_model):
    pos = np.arange(seq_len, dtype=np.float32)[:, None]
    i = np.arange(d_model // 2, dtype=np.float32)[None, :]
    div = np.exp(-(2.0 * i / d_model) * np.log(10000.0))
    ang = pos * div
    pe = np.zeros((seq_len, d_model), dtype=np.float32)
    pe[:, 0::2] = np.sin(ang)
    pe[:, 1::2] = np.cos(ang)
    return pe


def _pe_packed_words(seq_len, d_model):
    flat = _positional_encoding_np(seq_len, d_model).reshape(-1, 32)
    bf = flat.astype(jnp.bfloat16).view(np.uint16).astype(np.uint32)
    words = bf[:, :16] | (bf[:, 16:] << 16)
    return words.astype(np.int32).reshape(-1)


@functools.lru_cache(maxsize=None)
def _build(batch, seq, vocab, d):
    info = plsc.get_sparse_core_info()
    nc, ns = info.num_cores, info.num_subcores
    nw = nc * ns
    pos_per_w = seq // nw
    n_grp = pos_per_w // _CHUNK
    n_pair = d // (2 * _LANES)
    scale = float(np.sqrt(d))
    mesh = plsc.VectorSubcoreMesh(core_axis_name="c", subcore_axis_name="s")

    scratch = [
        pltpu.VMEM((batch, pos_per_w), jnp.int32),
        pltpu.VMEM((_NRING * batch, _CHUNK, d), jnp.float32),
        pltpu.VMEM((_CHUNK * d // 2,), jnp.int32),
        pltpu.VMEM((_CHUNK * d // 2,), jnp.int32),
    ] + [pltpu.SemaphoreType.DMA for _ in range(_NRING * 2 + 3)]

    @functools.partial(
        pl.kernel,
        mesh=mesh,
        out_type=jax.ShapeDtypeStruct((batch, seq, d), jnp.float32),
        scratch_types=scratch,
    )
    def emb_kernel(x_hbm, table_hbm, pe_hbm, out_hbm, idx_v, rows_v,
                   pe_v0, pe_v1, *sems):
        pe_bufs = (pe_v0, pe_v1)
        g_sem = sems[0:_NRING]
        st_sem = sems[_NRING:2 * _NRING]
        pe_sem = sems[2 * _NRING:2 * _NRING + 2]
        idx_sem = sems[2 * _NRING + 2]

        wid = lax.axis_index("s") * nc + lax.axis_index("c")
        pos_base = wid * pos_per_w

        def start_gathers(g):
            p = g % _NRING
            return [
                pltpu.async_copy(
                    table_hbm.at[idx_v.at[b, pl.ds(g * _CHUNK, _CHUNK)]],
                    rows_v.at[p * batch + b], g_sem[p])
                for b in range(batch)
            ]

        def start_pe(g):
            off = (pos_base + g * _CHUNK) * (d // 2)
            return pltpu.async_copy(
                pe_hbm.at[pl.ds(off, _CHUNK * d // 2)],
                pe_bufs[g % 2], pe_sem[g % 2])

        idx_hs = [
            pltpu.async_copy(x_hbm.at[b, pl.ds(pos_base, pos_per_w)],
                             idx_v.at[b], idx_sem)
            for b in range(batch)
        ]
        pe_h = {0: start_pe(0)}
        if n_grp > 1:
            pe_h[1] = start_pe(1)
        for h in idx_hs:
            h.wait()
        g_h = {0: start_gathers(0), 1: start_gathers(1)}

        st_h = {}
        for g in range(n_grp):
            p = g % _NRING
            q = g % 2
            if g + 2 < n_grp:
                if g - 1 >= 0:
                    for h in st_h[g - 1]:
                        h.wait()
                g_h[g + 2] = start_gathers(g + 2)
            pe_h[g].wait()
            for h in g_h[g]:
                h.wait()

            def row_body(r, _):
                row_base = r * d

                @plsc.parallel_loop(0, n_pair, 1, unroll=_CUNROLL)
                def pair_body(c):
                    off = c * (2 * _LANES)
                    pw = pe_bufs[q][pl.ds(row_base // 2 + c * _LANES, _LANES)]
                    pva = lax.bitcast_convert_type(
                        lax.shift_left(pw, 16), jnp.float32)
                    pvb = lax.bitcast_convert_type(
                        lax.bitwise_and(pw, jnp.int32(-65536)), jnp.float32)
                    for half, pv in ((0, pva), (1, pvb)):
                        o = off + half * _LANES
                        for b in range(batch):
                            k = p * batch + b
                            rv = rows_v[k, r, pl.ds(o, _LANES)]
                            rows_v[k, r, pl.ds(o, _LANES)] = rv * scale + pv

                return 0

            lax.fori_loop(0, _CHUNK, row_body, 0)

            hs = []
            for b in range(batch):
                hs.append(pltpu.async_copy(
                    rows_v.at[p * batch + b],
                    out_hbm.at[b, pl.ds(pos_base + g * _CHUNK, _CHUNK)],
                    st_sem[p]))
            st_h[g] = hs
            if g + 2 < n_grp:
                pe_h[g + 2] = start_pe(g + 2)

        for g in range(max(0, n_grp - _NRING), n_grp):
            for h in st_h[g]:
                h.wait()

    return emb_kernel


def kernel(x, table):
    b, s = x.shape
    v, d = table.shape
    pe = jnp.asarray(_pe_packed_words(s, d))
    return _build(b, s, v, d)(x, table, pe)
